# baseline (speedup 1.0000x reference)
import jax
import jax.numpy as jnp
from jax.experimental import pallas as pl
from jax.experimental.pallas import tpu as pltpu

_NP = 8
_B = _NP * 64


def _body(pairs, x_ref, out_ref, sem):
    copies = []
    for t in range(_NP):
        i1 = pairs[t]
        i2 = pairs[t + _NP]
        copies.append(
            pltpu.make_async_copy(
                x_ref.at[i1, :, i2, :], out_ref.at[pl.ds(t * 64, 64), :], sem
            )
        )
    for c in copies:
        c.start()
    for c in copies:
        c.wait()


_gather_tc = pl.pallas_call(
    _body,
    out_shape=jax.ShapeDtypeStruct((_B, 64), jnp.float32),
    grid_spec=pltpu.PrefetchScalarGridSpec(
        num_scalar_prefetch=1,
        grid=(1,),
        in_specs=[pl.BlockSpec(memory_space=pl.ANY)],
        out_specs=pl.BlockSpec(memory_space=pl.ANY),
        scratch_shapes=[pltpu.SemaphoreType.DMA],
    ),
)


def kernel(x, index1, index2):
    pairs = jnp.concatenate(
        [index1.reshape(8).astype(jnp.int32), index2.reshape(8).astype(jnp.int32)]
    )
    return _gather_tc(pairs, x).reshape(4, 2, 64, 64)


# speedup vs baseline: 15.1498x; 15.1498x over previous
import functools

import jax
import jax.numpy as jnp
from jax import lax
from jax.experimental import pallas as pl
from jax.experimental.pallas import tpu as pltpu
from jax.experimental.pallas import tpu_sc as plsc

_NC = 2
_NS = 16
_L = 16
_NW = _NC * _NS
_NP = 8
_B = _NP * 64
_BPW = _B // _NW
_EPW = _BPW * 64

_mesh = plsc.VectorSubcoreMesh(core_axis_name="c", subcore_axis_name="s")


@functools.partial(
    pl.kernel,
    mesh=_mesh,
    out_type=jax.ShapeDtypeStruct((_B * 64,), jnp.float32),
    scratch_types=[
        pltpu.VMEM((2 * _NP, _L), jnp.int32),
        pltpu.VMEM((_EPW,), jnp.int32),
        pltpu.VMEM((_EPW,), jnp.float32),
        pltpu.SemaphoreType.DMA,
    ],
)
def _gather_sc(x_hbm, pack_hbm, out_hbm, pack_v, idx_v, elems_v, sem):
    wid = lax.axis_index("s") * _NC + lax.axis_index("c")
    p = wid // 4
    jbase = (wid % 4) * _BPW
    _z = jnp.int32(0)
    _o = jnp.int32(1)
    pltpu.sync_copy(pack_hbm.at[p], pack_v.at[_z])
    pltpu.sync_copy(pack_hbm.at[p + _NP], pack_v.at[_o])
    i1v = pack_v[_z]
    i2v = pack_v[_o]
    lane = lax.iota(jnp.int32, _L)
    for c in range(_EPW // _L):
        j = jbase + c // 4
        lvec = (c % 4) * _L + lane
        idx_v[pl.ds(c * _L, _L)] = ((i1v * 64 + j) * 64 + lvec) * 128 + i2v
    pltpu.async_copy(x_hbm.at[idx_v], elems_v, sem).wait()
    pltpu.sync_copy(elems_v, out_hbm.at[pl.ds(wid * _EPW, _EPW)])


def kernel(x, index1, index2):
    xflat = x.transpose(0, 1, 3, 2).reshape(-1)
    pairs = jnp.concatenate(
        [index1.reshape(8).astype(jnp.int32), index2.reshape(8).astype(jnp.int32)]
    )
    pack = jnp.broadcast_to(pairs[:, None], (2 * _NP, _L))
    return _gather_sc(xflat, pack).reshape(4, 2, 64, 64)


# speedup vs baseline: 15.5834x; 1.0286x over previous
import functools

import jax
import jax.numpy as jnp
from jax import lax
from jax.experimental import pallas as pl
from jax.experimental.pallas import tpu as pltpu
from jax.experimental.pallas import tpu_sc as plsc

_NC = 2
_NS = 16
_L = 16
_NW = _NC * _NS
_NP = 8
_B = _NP * 64
_BPW = _B // _NW
_EPW = _BPW * 64

_mesh = plsc.VectorSubcoreMesh(core_axis_name="c", subcore_axis_name="s")


@functools.partial(
    pl.kernel,
    mesh=_mesh,
    out_type=jax.ShapeDtypeStruct((_B * 64,), jnp.float32),
    scratch_types=[
        pltpu.VMEM((2, _L), jnp.int32),
        pltpu.VMEM((_EPW,), jnp.int32),
        pltpu.VMEM((_EPW,), jnp.float32),
        pltpu.SemaphoreType.DMA,
    ],
)
def _gather_sc(x_hbm, pack_hbm, out_hbm, pair_v, idx_v, elems_v, sem):
    wid = lax.axis_index("s") * _NC + lax.axis_index("c")
    p = wid // 4
    jbase = (wid % 4) * _BPW
    _z = jnp.int32(0)
    _o = jnp.int32(1)
    pltpu.sync_copy(pack_hbm.at[p], pair_v)
    i1v = pair_v[_z]
    i2v = pair_v[_o]
    lane = lax.iota(jnp.int32, _L)
    base = ((i1v * 64 + jbase) * 64 + lane) * 128 + i2v

    def _chunk(j, carry):
        for q in range(4):
            idx_v[pl.ds((j * 4 + q) * _L, _L)] = base + (j * 64 + q * _L) * 128
        return carry

    lax.fori_loop(jnp.int32(0), jnp.int32(_BPW), _chunk, None)
    pltpu.async_copy(x_hbm.at[idx_v], elems_v, sem).wait()
    pltpu.sync_copy(elems_v, out_hbm.at[pl.ds(wid * _EPW, _EPW)])


def kernel(x, index1, index2):
    xflat = x.transpose(0, 1, 3, 2).reshape(-1)
    pairs = jnp.stack(
        [index1.reshape(8).astype(jnp.int32), index2.reshape(8).astype(jnp.int32)],
        axis=1,
    )
    pack = jnp.broadcast_to(pairs[:, :, None], (_NP, 2, _L))
    return _gather_sc(xflat, pack).reshape(4, 2, 64, 64)
